# baseline (speedup 1.0000x reference)
import functools

import jax

jax.config.update("jax_enable_x64", True)
import jax.numpy as jnp
from jax import lax
from jax.experimental import pallas as pl
from jax.experimental.pallas import tpu as pltpu
from jax.experimental.pallas import tpu_sc as plsc

def _Z():
    return jnp.int32(0)


N = 10240
E = 20480
FEAT = 32
NUM_RUNS = 2
NGRAPH = 512
N2 = NUM_RUNS * N
E2 = NUM_RUNS * E
NC, NS = 2, 16
NW = NC * NS
EPW = E2 // NW
CH = 128
NCH = EPW // CH
RPT = N2 // NS


def _mesh():
    return plsc.VectorSubcoreMesh(core_axis_name="c", subcore_axis_name="s")


@functools.lru_cache(maxsize=None)
def _gather_fn(cin):
    @functools.partial(
        pl.kernel,
        mesh=_mesh(),
        compiler_params=pltpu.CompilerParams(use_tc_tiling_on_sc=False),
        out_type=jax.ShapeDtypeStruct((E2, cin), jnp.float32),
        scratch_types=[
            pltpu.VMEM((NCH, CH), jnp.int32),
            pltpu.VMEM((EPW, cin), jnp.float32),
            pltpu.SemaphoreType.DMA,
        ],
    )
    def gk(src_hbm, table_hbm, xj_hbm, idx_v, rows_v, sem):
        w = lax.axis_index("c") * NS + lax.axis_index("s")
        pltpu.sync_copy(src_hbm.at[w], idx_v)

        def body(j, carry):
            pltpu.async_copy(
                table_hbm.at[idx_v.at[j]], rows_v.at[pl.ds(j * CH, CH)], sem
            ).wait()
            return carry

        lax.fori_loop(jnp.int32(0), jnp.int32(NCH), body, jnp.int32(0))
        pltpu.sync_copy(rows_v, xj_hbm.at[pl.ds(w * EPW, EPW)])

    return gk


@functools.lru_cache(maxsize=None)
def _scatter_fn(cout):
    NH = cout // 32

    @functools.partial(
        pl.kernel,
        mesh=_mesh(),
        compiler_params=pltpu.CompilerParams(use_tc_tiling_on_sc=False),
        out_type=jax.ShapeDtypeStruct((NC, NH, N2, 32), jnp.float32),
        scratch_types=[
            pltpu.VMEM((NCH, CH), jnp.int32),
            pltpu.VMEM((EPW, 32), jnp.float32),
            pltpu.VMEM_SHARED((N2, 32), jnp.float32),
        ],
    )
    def sk(dst_hbm, msg_hbm, zero_hbm, out_hbm, idx_v, msg_v, acc_sh):
        c = lax.axis_index("c")
        s = lax.axis_index("s")
        w = c * NS + s
        rs = s * RPT
        pltpu.sync_copy(dst_hbm.at[w], idx_v)
        for h in range(NH):
            pltpu.sync_copy(
                zero_hbm.at[pl.ds(rs, RPT)], acc_sh.at[pl.ds(rs, RPT)]
            )
            pltpu.sync_copy(
                msg_hbm.at[pl.ds(w * EPW, EPW), pl.ds(h * 32, 32)], msg_v
            )
            plsc.subcore_barrier()

            def body(j, carry):
                pltpu.sync_copy(
                    msg_v.at[pl.ds(j * CH, CH)], acc_sh.at[idx_v.at[j]], add=True
                )
                return carry

            lax.fori_loop(jnp.int32(0), jnp.int32(NCH), body, jnp.int32(0))
            plsc.subcore_barrier()
            pltpu.sync_copy(
                acc_sh.at[pl.ds(rs, RPT)],
                out_hbm.at[c, jnp.int32(h), pl.ds(rs, RPT)],
            )
            if h + 1 < NH:
                plsc.subcore_barrier()

    return sk


def _prep(x, keep):
    R = 2048
    G = N // R

    def body(x_ref, m_ref, out_ref):
        for r in range(NUM_RUNS):
            out_ref[r] = x_ref[...] * m_ref[r]

    return pl.pallas_call(
        body,
        grid=(G,),
        in_specs=[
            pl.BlockSpec((R, FEAT), lambda i: (i, _Z())),
            pl.BlockSpec((NUM_RUNS, R, 1), lambda i: (_Z(), i, _Z())),
        ],
        out_specs=pl.BlockSpec((NUM_RUNS, R, FEAT), lambda i: (_Z(), i, _Z())),
        out_shape=jax.ShapeDtypeStruct((NUM_RUNS, N, FEAT), jnp.float32),
    )(x, keep)


def _msgs(ea, xj2, W1, b1, W2, b2, T, cin, cout):
    B = 256
    G = E // B

    def body(ea_ref, xj_ref, W1_ref, b1_ref, W2_ref, b2_ref, T_ref, out_ref):
        eh = jnp.maximum(
            jnp.dot(ea_ref[...], W1_ref[...], preferred_element_type=jnp.float32)
            + b1_ref[...][None, :],
            0.0,
        )
        wef = (
            jnp.dot(eh, W2_ref[...], preferred_element_type=jnp.float32)
            + b2_ref[...][None, :]
        )
        for r in range(NUM_RUNS):
            xh = xj_ref[r].astype(jnp.bfloat16).astype(jnp.float32)
            xl = xj_ref[r] - xh
            xb = jnp.dot(
                xh, T_ref[...], preferred_element_type=jnp.float32
            ) + jnp.dot(
                xl, T_ref[...], preferred_element_type=jnp.float32
            )
            p = xb * wef
            w = cin * cout
            while w > cout:
                p = p[:, : w // 2] + p[:, w // 2:]
                w //= 2
            out_ref[r] = p

    return pl.pallas_call(
        body,
        grid=(G,),
        in_specs=[
            pl.BlockSpec((B, 5), lambda i: (i, _Z())),
            pl.BlockSpec((NUM_RUNS, B, cin), lambda i: (_Z(), i, _Z())),
            pl.BlockSpec((5, 128), lambda i: (_Z(), _Z())),
            pl.BlockSpec((128,), lambda i: (_Z(),)),
            pl.BlockSpec((128, cin * cout), lambda i: (_Z(), _Z())),
            pl.BlockSpec((cin * cout,), lambda i: (_Z(),)),
            pl.BlockSpec((cin, cin * cout), lambda i: (_Z(), _Z())),
        ],
        out_specs=pl.BlockSpec((NUM_RUNS, B, cout), lambda i: (_Z(), i, _Z())),
        out_shape=jax.ShapeDtypeStruct((NUM_RUNS, E, cout), jnp.float32),
        compiler_params=pltpu.CompilerParams(
            dimension_semantics=("arbitrary",)
        ),
    )(ea, xj2, W1, b1, W2, b2, T)


def _combine(t, parts, root, bias, cin, cout):
    R = 2048
    G = N2 // R
    NH = cout // 32

    def body(t_ref, p_ref, root_ref, bias_ref, out_ref):
        agg = [p_ref[0, h] + p_ref[1, h] for h in range(NH)]
        agg = agg[0] if NH == 1 else jnp.concatenate(agg, axis=-1)
        v = (
            agg
            + jnp.dot(t_ref[...], root_ref[...], preferred_element_type=jnp.float32)
            + bias_ref[...][None, :]
        )
        out_ref[...] = jnp.where(v > 0, v, jnp.exp(v) - 1.0)

    return pl.pallas_call(
        body,
        grid=(G,),
        in_specs=[
            pl.BlockSpec((R, cin), lambda i: (i, _Z())),
            pl.BlockSpec((NC, NH, R, 32), lambda i: (_Z(), _Z(), i, _Z())),
            pl.BlockSpec((cin, cout), lambda i: (_Z(), _Z())),
            pl.BlockSpec((cout,), lambda i: (_Z(),)),
        ],
        out_specs=pl.BlockSpec((R, cout), lambda i: (i, _Z())),
        out_shape=jax.ShapeDtypeStruct((N2, cout), jnp.float32),
    )(t, parts, root, bias)


def _final(h2, batch3, fc1_W, fc1_b, fc2_W, fc2_b, fc3_W, fc3_b):
    R = 2048
    G = N // R

    def body(h_ref, b_ref, w1, v1, w2, v2, w3, v3, out_ref, acc, cnt):
        i = pl.program_id(0)

        @pl.when(i == 0)
        def _():
            acc[...] = jnp.zeros_like(acc)
            cnt[...] = jnp.zeros_like(cnt)

        m = 0.5 * (h_ref[0] + h_ref[1])
        bids = b_ref[0, 0, :]
        rows = lax.broadcasted_iota(jnp.int32, (NGRAPH, R), 0)
        oh = (rows == bids[None, :]).astype(jnp.float32)
        acc[...] += jnp.dot(oh, m, preferred_element_type=jnp.float32)
        cnt[...] += jnp.sum(oh, axis=1, keepdims=True)

        @pl.when(i == G - 1)
        def _():
            g = acc[...] / jnp.maximum(cnt[...], 1.0)
            g = jnp.dot(g, w1[...], preferred_element_type=jnp.float32) + v1[...][None, :]
            g = jnp.where(g > 0, g, jnp.exp(g) - 1.0)
            g = jnp.dot(g, w2[...], preferred_element_type=jnp.float32) + v2[...][None, :]
            g = jnp.where(g > 0, g, jnp.exp(g) - 1.0)
            out_ref[...] = (
                jnp.dot(g, w3[...], preferred_element_type=jnp.float32)
                + v3[...][None, :]
            )

    full = lambda *s: pl.BlockSpec(s, lambda i: tuple(_Z() for _ in s))
    return pl.pallas_call(
        body,
        grid=(G,),
        in_specs=[
            pl.BlockSpec((NUM_RUNS, R, 64), lambda i: (_Z(), i, _Z())),
            pl.BlockSpec((1, 1, R), lambda i: (i, _Z(), _Z())),
            full(64, 32),
            full(32),
            full(32, 16),
            full(16),
            full(16, 1),
            full(1),
        ],
        out_specs=pl.BlockSpec((NGRAPH, 1), lambda i: (_Z(), _Z())),
        out_shape=jax.ShapeDtypeStruct((NGRAPH, 1), jnp.float32),
        scratch_shapes=[
            pltpu.VMEM((NGRAPH, 64), jnp.float32),
            pltpu.VMEM((NGRAPH, 1), jnp.float32),
        ],
        compiler_params=pltpu.CompilerParams(
            dimension_semantics=("arbitrary",)
        ),
    )(h2, batch3, fc1_W, fc1_b, fc2_W, fc2_b, fc3_W, fc3_b)


def kernel(x, edge_index, edge_attr, batch,
           nn1_W1, nn1_b1, nn1_W2, nn1_b2, root1, bias1,
           nn2_W1, nn2_b1, nn2_W2, nn2_b2, root2, bias2,
           nn3_W1, nn3_b1, nn3_W2, nn3_b2, root3, bias3,
           fc1_W, fc1_b, fc2_W, fc2_b, fc3_W, fc3_b):
    f32 = jnp.float32
    (nn1_W1, nn1_b1, nn1_W2, nn1_b2, root1, bias1,
     nn2_W1, nn2_b1, nn2_W2, nn2_b2, root2, bias2,
     nn3_W1, nn3_b1, nn3_W2, nn3_b2, root3, bias3,
     fc1_W, fc1_b, fc2_W, fc2_b, fc3_W, fc3_b) = jax.tree.map(
        lambda a: a.astype(f32),
        (nn1_W1, nn1_b1, nn1_W2, nn1_b2, root1, bias1,
         nn2_W1, nn2_b1, nn2_W2, nn2_b2, root2, bias2,
         nn3_W1, nn3_b1, nn3_W2, nn3_b2, root3, bias3,
         fc1_W, fc1_b, fc2_W, fc2_b, fc3_W, fc3_b))
    x = x.astype(jnp.float32)
    ea = edge_attr.astype(jnp.float32)
    ei = edge_index.astype(jnp.int32)
    off = jnp.max(ei) + 1
    src2 = jnp.concatenate([ei[0], ei[0] + off]).reshape(NW, NCH, CH)
    dst2 = jnp.concatenate([ei[1], ei[1] + off]).reshape(NW, NCH, CH)

    drop = jax.random.bernoulli(
        jax.random.key(42), 2.0 / (1.0 + NUM_RUNS), (NUM_RUNS, N)
    )
    keep = jnp.where(drop, 0.0, 1.0).astype(jnp.float32)[..., None]

    t = _prep(x, keep).reshape(N2, FEAT)

    layers = [
        (nn1_W1, nn1_b1, nn1_W2, nn1_b2, root1, bias1, FEAT, 32),
        (nn2_W1, nn2_b1, nn2_W2, nn2_b2, root2, bias2, 32, 64),
        (nn3_W1, nn3_b1, nn3_W2, nn3_b2, root3, bias3, 64, 64),
    ]
    for W1, b1, W2, b2, root, bias, cin, cout in layers:
        xj = _gather_fn(cin)(src2, t)
        T = jnp.repeat(jnp.eye(cin, dtype=jnp.float32), cout, axis=1)
        msg = _msgs(ea, xj.reshape(NUM_RUNS, E, cin), W1, b1, W2, b2, T,
                    cin, cout)
        zeros = jnp.zeros((N2, 32), jnp.float32)
        parts = _scatter_fn(cout)(dst2, msg.reshape(E2, cout), zeros)
        t = _combine(t, parts, root, bias, cin, cout)

    batch3 = batch.astype(jnp.int32).reshape(N // 2048, 1, 2048)
    g = _final(t.reshape(NUM_RUNS, N, 64), batch3,
               fc1_W, fc1_b, fc2_W, fc2_b, fc3_W, fc3_b)
    return g.reshape(-1).astype(jnp.float64)


# speedup vs baseline: 1.2303x; 1.2303x over previous
import functools

import jax

jax.config.update("jax_enable_x64", True)
import jax.numpy as jnp
from jax import lax
from jax.experimental import pallas as pl
from jax.experimental.pallas import tpu as pltpu
from jax.experimental.pallas import tpu_sc as plsc

def _Z():
    return jnp.int32(0)


N = 10240
E = 20480
FEAT = 32
NUM_RUNS = 2
NGRAPH = 512
N2 = NUM_RUNS * N
E2 = NUM_RUNS * E
NC, NS = 2, 16
NW = NC * NS
EPW = E2 // NW
CH = 128
NCH = EPW // CH
RPT = N2 // NS


def _mesh():
    return plsc.VectorSubcoreMesh(core_axis_name="c", subcore_axis_name="s")


@functools.lru_cache(maxsize=None)
def _gather_fn(cin):
    @functools.partial(
        pl.kernel,
        mesh=_mesh(),
        compiler_params=pltpu.CompilerParams(use_tc_tiling_on_sc=False),
        out_type=jax.ShapeDtypeStruct((E2, cin), jnp.float32),
        scratch_types=[
            pltpu.VMEM((NCH, CH), jnp.int32),
            pltpu.VMEM((EPW, cin), jnp.float32),
            pltpu.SemaphoreType.DMA,
        ],
    )
    def gk(src_hbm, table_hbm, xj_hbm, idx_v, rows_v, sem):
        w = lax.axis_index("c") * NS + lax.axis_index("s")
        pltpu.sync_copy(src_hbm.at[w], idx_v)
        handles = [
            pltpu.async_copy(
                table_hbm.at[idx_v.at[jnp.int32(j)]],
                rows_v.at[pl.ds(j * CH, CH)],
                sem,
            )
            for j in range(NCH)
        ]
        for h in handles:
            h.wait()
        pltpu.sync_copy(rows_v, xj_hbm.at[pl.ds(w * EPW, EPW)])

    return gk


EPT = E2 // NS
NCT = EPT // CH


@functools.lru_cache(maxsize=None)
def _scatter_fn(cout):
    CPC = cout // NC

    @functools.partial(
        pl.kernel,
        mesh=_mesh(),
        compiler_params=pltpu.CompilerParams(use_tc_tiling_on_sc=False),
        out_type=jax.ShapeDtypeStruct((N2, cout), jnp.float32),
        scratch_types=[
            pltpu.VMEM((NCT, CH), jnp.int32),
            pltpu.VMEM((EPT, CPC), jnp.float32),
            pltpu.VMEM_SHARED((N2, CPC), jnp.float32),
            pltpu.SemaphoreType.DMA,
        ],
    )
    def sk(dst_hbm, msg_hbm, zero_hbm, out_hbm, idx_v, msg_v, acc_sh, sem):
        c = lax.axis_index("c")
        s = lax.axis_index("s")
        rs = s * RPT
        pltpu.sync_copy(dst_hbm.at[s], idx_v)
        pltpu.sync_copy(zero_hbm.at[pl.ds(rs, RPT)], acc_sh.at[pl.ds(rs, RPT)])
        pltpu.sync_copy(
            msg_hbm.at[pl.ds(s * EPT, EPT), pl.ds(c * CPC, CPC)], msg_v
        )
        plsc.subcore_barrier()
        def body(j, carry):
            pltpu.sync_copy(
                msg_v.at[pl.ds(j * CH, CH)], acc_sh.at[idx_v.at[j]], add=True
            )
            return carry

        lax.fori_loop(jnp.int32(0), jnp.int32(NCT), body, jnp.int32(0))
        plsc.subcore_barrier()
        pltpu.sync_copy(
            acc_sh.at[pl.ds(rs, RPT)],
            out_hbm.at[pl.ds(rs, RPT), pl.ds(c * CPC, CPC)],
        )

    return sk


def _prep(x, keep):
    R = 2048
    G = N // R

    def body(x_ref, m_ref, out_ref):
        for r in range(NUM_RUNS):
            out_ref[r] = x_ref[...] * m_ref[r]

    return pl.pallas_call(
        body,
        grid=(G,),
        in_specs=[
            pl.BlockSpec((R, FEAT), lambda i: (i, _Z())),
            pl.BlockSpec((NUM_RUNS, R, 1), lambda i: (_Z(), i, _Z())),
        ],
        out_specs=pl.BlockSpec((NUM_RUNS, R, FEAT), lambda i: (_Z(), i, _Z())),
        out_shape=jax.ShapeDtypeStruct((NUM_RUNS, N, FEAT), jnp.float32),
    )(x, keep)


def _msgs(ea, xj2, W1, b1, W2, b2, T, cin, cout):
    B = 256
    G = E // B

    def body(ea_ref, xj_ref, W1_ref, b1_ref, W2_ref, b2_ref, T_ref, out_ref):
        eh = jnp.maximum(
            jnp.dot(ea_ref[...], W1_ref[...], preferred_element_type=jnp.float32)
            + b1_ref[...][None, :],
            0.0,
        )
        wef = (
            jnp.dot(eh, W2_ref[...], preferred_element_type=jnp.float32)
            + b2_ref[...][None, :]
        )
        for r in range(NUM_RUNS):
            xh = xj_ref[r].astype(jnp.bfloat16).astype(jnp.float32)
            xl = xj_ref[r] - xh
            xb = jnp.dot(
                jnp.concatenate([xh, xl], axis=1),
                T_ref[...],
                preferred_element_type=jnp.float32,
            )
            p = xb * wef
            w = cin * cout
            while w > cout:
                p = p[:, : w // 2] + p[:, w // 2:]
                w //= 2
            out_ref[r] = p

    return pl.pallas_call(
        body,
        grid=(G,),
        in_specs=[
            pl.BlockSpec((B, 5), lambda i: (i, _Z())),
            pl.BlockSpec((NUM_RUNS, B, cin), lambda i: (_Z(), i, _Z())),
            pl.BlockSpec((5, 128), lambda i: (_Z(), _Z())),
            pl.BlockSpec((128,), lambda i: (_Z(),)),
            pl.BlockSpec((128, cin * cout), lambda i: (_Z(), _Z())),
            pl.BlockSpec((cin * cout,), lambda i: (_Z(),)),
            pl.BlockSpec((2 * cin, cin * cout), lambda i: (_Z(), _Z())),
        ],
        out_specs=pl.BlockSpec((NUM_RUNS, B, cout), lambda i: (_Z(), i, _Z())),
        out_shape=jax.ShapeDtypeStruct((NUM_RUNS, E, cout), jnp.float32),
        compiler_params=pltpu.CompilerParams(
            dimension_semantics=("arbitrary",)
        ),
    )(ea, xj2, W1, b1, W2, b2, T)


def _combine(t, agg, root, bias, cin, cout):
    R = 2048
    G = N2 // R

    def body(t_ref, a_ref, root_ref, bias_ref, out_ref):
        v = (
            a_ref[...]
            + jnp.dot(t_ref[...], root_ref[...], preferred_element_type=jnp.float32)
            + bias_ref[...][None, :]
        )
        out_ref[...] = jnp.where(v > 0, v, jnp.exp(v) - 1.0)

    return pl.pallas_call(
        body,
        grid=(G,),
        in_specs=[
            pl.BlockSpec((R, cin), lambda i: (i, _Z())),
            pl.BlockSpec((R, cout), lambda i: (i, _Z())),
            pl.BlockSpec((cin, cout), lambda i: (_Z(), _Z())),
            pl.BlockSpec((cout,), lambda i: (_Z(),)),
        ],
        out_specs=pl.BlockSpec((R, cout), lambda i: (i, _Z())),
        out_shape=jax.ShapeDtypeStruct((N2, cout), jnp.float32),
    )(t, agg, root, bias)


def _final(h2, batch3, fc1_W, fc1_b, fc2_W, fc2_b, fc3_W, fc3_b):
    R = 2048
    G = N // R

    def body(h_ref, b_ref, w1, v1, w2, v2, w3, v3, out_ref, acc, cnt):
        i = pl.program_id(0)

        @pl.when(i == 0)
        def _():
            acc[...] = jnp.zeros_like(acc)
            cnt[...] = jnp.zeros_like(cnt)

        m = 0.5 * (h_ref[0] + h_ref[1])
        bids = b_ref[0, 0, :]
        rows = lax.broadcasted_iota(jnp.int32, (NGRAPH, R), 0)
        oh = (rows == bids[None, :]).astype(jnp.float32)
        acc[...] += jnp.dot(oh, m, preferred_element_type=jnp.float32)
        cnt[...] += jnp.sum(oh, axis=1, keepdims=True)

        @pl.when(i == G - 1)
        def _():
            g = acc[...] / jnp.maximum(cnt[...], 1.0)
            g = jnp.dot(g, w1[...], preferred_element_type=jnp.float32) + v1[...][None, :]
            g = jnp.where(g > 0, g, jnp.exp(g) - 1.0)
            g = jnp.dot(g, w2[...], preferred_element_type=jnp.float32) + v2[...][None, :]
            g = jnp.where(g > 0, g, jnp.exp(g) - 1.0)
            out_ref[...] = (
                jnp.dot(g, w3[...], preferred_element_type=jnp.float32)
                + v3[...][None, :]
            )

    full = lambda *s: pl.BlockSpec(s, lambda i: tuple(_Z() for _ in s))
    return pl.pallas_call(
        body,
        grid=(G,),
        in_specs=[
            pl.BlockSpec((NUM_RUNS, R, 64), lambda i: (_Z(), i, _Z())),
            pl.BlockSpec((1, 1, R), lambda i: (i, _Z(), _Z())),
            full(64, 32),
            full(32),
            full(32, 16),
            full(16),
            full(16, 1),
            full(1),
        ],
        out_specs=pl.BlockSpec((NGRAPH, 1), lambda i: (_Z(), _Z())),
        out_shape=jax.ShapeDtypeStruct((NGRAPH, 1), jnp.float32),
        scratch_shapes=[
            pltpu.VMEM((NGRAPH, 64), jnp.float32),
            pltpu.VMEM((NGRAPH, 1), jnp.float32),
        ],
        compiler_params=pltpu.CompilerParams(
            dimension_semantics=("arbitrary",)
        ),
    )(h2, batch3, fc1_W, fc1_b, fc2_W, fc2_b, fc3_W, fc3_b)


def kernel(x, edge_index, edge_attr, batch,
           nn1_W1, nn1_b1, nn1_W2, nn1_b2, root1, bias1,
           nn2_W1, nn2_b1, nn2_W2, nn2_b2, root2, bias2,
           nn3_W1, nn3_b1, nn3_W2, nn3_b2, root3, bias3,
           fc1_W, fc1_b, fc2_W, fc2_b, fc3_W, fc3_b):
    f32 = jnp.float32
    (nn1_W1, nn1_b1, nn1_W2, nn1_b2, root1, bias1,
     nn2_W1, nn2_b1, nn2_W2, nn2_b2, root2, bias2,
     nn3_W1, nn3_b1, nn3_W2, nn3_b2, root3, bias3,
     fc1_W, fc1_b, fc2_W, fc2_b, fc3_W, fc3_b) = jax.tree.map(
        lambda a: a.astype(f32),
        (nn1_W1, nn1_b1, nn1_W2, nn1_b2, root1, bias1,
         nn2_W1, nn2_b1, nn2_W2, nn2_b2, root2, bias2,
         nn3_W1, nn3_b1, nn3_W2, nn3_b2, root3, bias3,
         fc1_W, fc1_b, fc2_W, fc2_b, fc3_W, fc3_b))
    x = x.astype(jnp.float32)
    ea = edge_attr.astype(jnp.float32)
    ei = edge_index.astype(jnp.int32)
    off = jnp.max(ei) + 1
    src2 = jnp.concatenate([ei[0], ei[0] + off]).reshape(NW, NCH, CH)
    dst2 = jnp.concatenate([ei[1], ei[1] + off]).reshape(NS, NCT, CH)

    drop = jax.random.bernoulli(
        jax.random.key(42), 2.0 / (1.0 + NUM_RUNS), (NUM_RUNS, N)
    )
    keep = jnp.where(drop, 0.0, 1.0).astype(jnp.float32)[..., None]

    t = _prep(x, keep).reshape(N2, FEAT)

    layers = [
        (nn1_W1, nn1_b1, nn1_W2, nn1_b2, root1, bias1, FEAT, 32),
        (nn2_W1, nn2_b1, nn2_W2, nn2_b2, root2, bias2, 32, 64),
        (nn3_W1, nn3_b1, nn3_W2, nn3_b2, root3, bias3, 64, 64),
    ]
    for W1, b1, W2, b2, root, bias, cin, cout in layers:
        xj = _gather_fn(cin)(src2, t)
        T1 = jnp.repeat(jnp.eye(cin, dtype=jnp.float32), cout, axis=1)
        T = jnp.concatenate([T1, T1], axis=0)
        msg = _msgs(ea, xj.reshape(NUM_RUNS, E, cin), W1, b1, W2, b2, T,
                    cin, cout)
        zeros = jnp.zeros((N2, cout // NC), jnp.float32)
        agg = _scatter_fn(cout)(dst2, msg.reshape(E2, cout), zeros)
        t = _combine(t, agg, root, bias, cin, cout)

    batch3 = batch.astype(jnp.int32).reshape(N // 2048, 1, 2048)
    g = _final(t.reshape(NUM_RUNS, N, 64), batch3,
               fc1_W, fc1_b, fc2_W, fc2_b, fc3_W, fc3_b)
    return g.reshape(-1).astype(jnp.float64)
